# baseline (speedup 1.0000x reference)
import functools

import jax
import jax.numpy as jnp
from jax import lax
from jax.experimental import pallas as pl
from jax.experimental.pallas import tpu as pltpu
from jax.experimental.pallas import tpu_sc as plsc

N_TOK = 4096
D_IN = 768
D_DICT = 30720
K = 32
CHUNK = 128
N_CHUNKS = D_DICT // CHUNK
BLK_J = 512
L = 16

N_WORKERS = 32
ROWS_PER_W = N_TOK // N_WORKERS
CAND_CAP = D_DICT + 2 * L
SEL_CAP = 256


def _encode_body(nrows, x_ref, we_ref, be_ref, acts_ref, cm_ref):
    xc = x_ref[...]
    pre = jax.lax.dot_general(
        xc, we_ref[...], (((1,), (0,)), ((), ())),
        preferred_element_type=jnp.float32,
    )
    acts = jnp.maximum(pre + be_ref[...], 0.0)
    acts_ref[...] = acts
    cm = jnp.max(acts.reshape(nrows, BLK_J // CHUNK, CHUNK), axis=2)
    cm_ref[...] = cm.reshape(1, nrows, BLK_J // CHUNK)


def _encode(x_cent, W_enc, b_enc, nrows):
    nj = D_DICT // BLK_J
    return pl.pallas_call(
        functools.partial(_encode_body, nrows),
        grid=(nj,),
        in_specs=[
            pl.BlockSpec((nrows, D_IN), lambda j: (0, 0)),
            pl.BlockSpec((D_IN, BLK_J), lambda j: (0, j)),
            pl.BlockSpec((1, BLK_J), lambda j: (0, j)),
        ],
        out_specs=[
            pl.BlockSpec((nrows, BLK_J), lambda j: (0, j)),
            pl.BlockSpec((1, nrows, BLK_J // CHUNK), lambda j: (j, 0, 0)),
        ],
        out_shape=[
            jax.ShapeDtypeStruct((nrows, D_DICT), jnp.float32),
            jax.ShapeDtypeStruct((D_DICT // BLK_J, nrows, BLK_J // CHUNK),
                                 jnp.float32),
        ],
        compiler_params=pltpu.CompilerParams(
            dimension_semantics=("arbitrary",),
        ),
    )(x_cent, W_enc, b_enc)


def _iota16():
    return lax.iota(jnp.int32, L)


def _scal(x):
    return x if x.ndim == 0 else jnp.max(x)


def _extract_i32(vec, lane):
    return jnp.sum(jnp.where(_iota16() == lane, vec, 0))


def _extract_f32(vec, lane):
    return jnp.sum(jnp.where(_iota16() == lane, vec, jnp.float32(0)))


def _sc_body(rows_per_w, acts2d, cm, wdec, out,
             cma_v, cmb_v, cidx_v, cidx2d_v, chunk_v, cval_v, cgi_v,
             selva_v, selvb_v, selia_v, selib_v, wrowsa_v, wrowsb_v,
             outa_v, outb_v, sem, sem2, semc, semo):
    w = lax.axis_index("s") * 2 + lax.axis_index("c")
    lane0 = _iota16() == 0
    r0 = w * rows_per_w
    inf_v = jnp.full((L,), jnp.inf, jnp.float32)
    pltpu.sync_copy(cm.at[r0], cma_v)

    ga = (selva_v, selia_v, wrowsa_v, outa_v)
    gb = (selvb_v, selib_v, wrowsb_v, outb_v)

    def wdec_gather(g):
        selv, seli, wrows, _ = g
        return pltpu.make_async_copy(wdec.at[seli.at[pl.ds(0, K)]], wrows,
                                     sem2)

    def decode_row(g, rt):
        selv, seli, wrows, outb = g
        wdec_gather(g).wait()

        def dec_step(j, acc):
            jg = (j // L) * L
            val = _extract_f32(selv[pl.ds(jg, L)], j - jg)
            vvec = jnp.full((L,), val, jnp.float32)
            return tuple(acc[u] + vvec * wrows[j, pl.ds(u * L, L)]
                         for u in range(D_IN // L))

        acc0 = tuple(jnp.zeros((L,), jnp.float32) for _ in range(D_IN // L))
        acc = lax.fori_loop(0, K, dec_step, acc0)
        for u in range(D_IN // L):
            outb[pl.ds(u * L, L)] = acc[u]
        pltpu.async_copy(outb, out.at[rt], semo)

    def one_row(rl, cm_cur, cm_nxt, g_cur, g_prv):
        r = r0 + rl
        rn = r0 + jnp.minimum(rl + 1, rows_per_w - 1)
        pltpu.async_copy(cm.at[rn], cm_nxt, semc)
        cms = [cm_cur[pl.ds(i * L, L)] for i in range(N_CHUNKS // L)]

        def search_step(i, sbits):
            trial = jnp.bitwise_or(sbits, jnp.left_shift(jnp.int32(1), 30 - i))
            tvec = jnp.full((L,), lax.bitcast_convert_type(trial, jnp.float32),
                            jnp.float32)
            cnt = jnp.zeros((L,), jnp.int32)
            for c in cms:
                cnt = cnt + jnp.where(c >= tvec, 1, 0)
            return jnp.where(jnp.sum(cnt) >= K, trial, sbits)

        sbits = lax.fori_loop(0, 20, search_step, jnp.int32(0))
        svec = jnp.full((L,), lax.bitcast_convert_type(sbits, jnp.float32),
                        jnp.float32)

        base = r * N_CHUNKS
        nsel = jnp.int32(0)
        for i, c in enumerate(cms):
            m = c >= svec
            ids = base + i * L + _iota16()
            plsc.store_compressed(cidx_v.at[pl.ds(nsel, L)], ids, mask=m)
            nsel = nsel + jnp.sum(jnp.where(m, 1, 0))
        for i in range(4):
            cidx_v[pl.ds(nsel + i * L, L)] = base + i * L + _iota16()

        @pl.when(nsel > 64)
        def _():
            for i in range(SEL_CAP // L):
                ids = jnp.minimum(i * L + _iota16(), N_CHUNKS - 1)
                cidx_v[pl.ds(i * L, L)] = base + ids

        nsel2 = jnp.where(nsel > 64, jnp.int32(N_CHUNKS), nsel)
        ngrp = (nsel2 + 63) // 64

        for i in range(SEL_CAP // L):
            cidx2d_v[i // 4, pl.ds((i % 4) * L, L)] = cidx_v[pl.ds(i * L, L)]

        def extract_grp(g, cnt):
            pltpu.async_copy(acts2d.at[cidx2d_v.at[g]], chunk_v, sem).wait()
            kmax = jnp.minimum(nsel2 - g * 64, 64)

            def extract_slot(k, cnt):
                kglob = g * 64 + k
                kg = (kglob // L) * L
                cvec = cidx_v[pl.ds(kg, L)]
                cloc = _extract_i32(cvec, kglob - kg) - base
                gbase = jnp.full((L,), cloc * CHUNK, jnp.int32)
                for u in range(CHUNK // L):
                    v = chunk_v[k, pl.ds(u * L, L)]
                    m = v >= svec
                    gi = gbase + (u * L) + _iota16()
                    plsc.store_compressed(cval_v.at[pl.ds(cnt, L)], v, mask=m)
                    plsc.store_compressed(cgi_v.at[pl.ds(cnt, L)], gi, mask=m)
                    cnt = cnt + jnp.sum(jnp.where(m, 1, 0))
                return cnt

            return lax.fori_loop(0, kmax, extract_slot, cnt)

        cnt = lax.fori_loop(0, ngrp, extract_grp, jnp.int32(0))
        cval_v[pl.ds(cnt, L)] = inf_v
        cval_v[pl.ds(cnt + L, L)] = inf_v
        nv = jnp.maximum((cnt + L - 1) // L, 2)

        def discard_step(_, __):
            def colmin_step(v, acc):
                return jnp.minimum(acc, cval_v[pl.ds(v * L, L)])

            colmin = lax.fori_loop(1, nv, colmin_step, cval_v[pl.ds(0, L)])
            t = jnp.min(colmin)
            tvec = jnp.full((L,), t, jnp.float32)

            def find_cond(st):
                v, found, _ = st
                return jnp.logical_and(v >= 0, found == 0)

            def find_step(st):
                v, found, pos = st
                m = cval_v[pl.ds(v * L, L)] <= tvec
                hit = jnp.sum(jnp.where(m, 1, 0)) > 0
                lastlane = jnp.max(jnp.where(m, _iota16(), -1))
                pos = jnp.where(hit, v * L + lastlane, pos)
                return v - 1, jnp.where(hit, 1, found).astype(jnp.int32), pos

            _, _, pos = lax.while_loop(
                find_cond, find_step, (nv - 1, jnp.int32(0), jnp.int32(0)))
            plsc.store_scatter(cval_v, [jnp.full((L,), pos, jnp.int32)],
                               inf_v, mask=lane0)
            return 0

        lax.fori_loop(0, cnt - K, discard_step, 0)

        selv, seli, _, _ = g_cur

        def collect_step(v, scnt):
            vals = cval_v[pl.ds(v * L, L)]
            m = vals < inf_v
            plsc.store_compressed(selv.at[pl.ds(scnt, L)], vals, mask=m)
            plsc.store_compressed(seli.at[pl.ds(scnt, L)],
                                  cgi_v[pl.ds(v * L, L)], mask=m)
            return scnt + jnp.sum(jnp.where(m, 1, 0))

        lax.fori_loop(0, nv, collect_step, jnp.int32(0))
        wdec_gather(g_cur).start()

        @pl.when(rl >= 3)
        def _():
            pltpu.make_async_copy(g_prv[3], out.at[r], semo).wait()

        @pl.when(rl >= 1)
        def _():
            decode_row(g_prv, r - 1)

        pltpu.make_async_copy(cm.at[rn], cm_nxt, semc).wait()

    def pair_body(rp, _):
        rl = rp * 2
        one_row(rl, cma_v, cmb_v, ga, gb)
        one_row(rl + 1, cmb_v, cma_v, gb, ga)
        return 0

    lax.fori_loop(0, rows_per_w // 2, pair_body, 0)
    pltpu.make_async_copy(gb[3], out.at[r0], semo).wait()
    decode_row(gb, r0 + rows_per_w - 1)
    pltpu.make_async_copy(ga[3], out.at[r0], semo).wait()
    pltpu.make_async_copy(gb[3], out.at[r0], semo).wait()


@functools.partial(jax.jit, static_argnames=("nrows",))
def _sc_select_decode(acts2d, cm, W_dec, nrows=N_TOK):
    kern = pl.kernel(
        functools.partial(_sc_body, nrows // N_WORKERS),
        out_type=jax.ShapeDtypeStruct((nrows, D_IN), jnp.float32),
        mesh=plsc.VectorSubcoreMesh(core_axis_name="c", subcore_axis_name="s"),
        scratch_types=[
            pltpu.VMEM((N_CHUNKS,), jnp.float32),
            pltpu.VMEM((N_CHUNKS,), jnp.float32),
            pltpu.VMEM((SEL_CAP + L,), jnp.int32),
            pltpu.VMEM((4, 64), jnp.int32),
            pltpu.VMEM((64, CHUNK), jnp.float32),
            pltpu.VMEM((CAND_CAP,), jnp.float32),
            pltpu.VMEM((CAND_CAP,), jnp.int32),
            pltpu.VMEM((K + L,), jnp.float32),
            pltpu.VMEM((K + L,), jnp.float32),
            pltpu.VMEM((K + L,), jnp.int32),
            pltpu.VMEM((K + L,), jnp.int32),
            pltpu.VMEM((K, D_IN), jnp.float32),
            pltpu.VMEM((K, D_IN), jnp.float32),
            pltpu.VMEM((D_IN,), jnp.float32),
            pltpu.VMEM((D_IN,), jnp.float32),
            pltpu.SemaphoreType.DMA,
            pltpu.SemaphoreType.DMA,
            pltpu.SemaphoreType.DMA,
            pltpu.SemaphoreType.DMA,
        ],
        compiler_params=pltpu.CompilerParams(needs_layout_passes=False),
    )
    return kern(acts2d, cm, W_dec)


N_PIPE = 8
ROWS_BLK = N_TOK // N_PIPE


def kernel(x, W_enc, W_dec, b_enc, b_dec):
    x_cent = x - b_dec[None, :]
    b_enc2 = b_enc.reshape(1, D_DICT)
    recons = []
    for b in range(N_PIPE):
        xb = lax.dynamic_slice_in_dim(x_cent, b * ROWS_BLK, ROWS_BLK, axis=0)
        acts, cm3 = _encode(xb, W_enc, b_enc2, ROWS_BLK)
        cm = cm3.transpose(1, 0, 2).reshape(ROWS_BLK, N_CHUNKS)
        acts2d = acts.reshape(ROWS_BLK * N_CHUNKS, CHUNK)
        recons.append(_sc_select_decode(acts2d, cm, W_dec, nrows=ROWS_BLK))
    return jnp.concatenate(recons, axis=0) + b_dec[None, :]


# speedup vs baseline: 1.1296x; 1.1296x over previous
import functools

import jax
import jax.numpy as jnp
from jax import lax
from jax.experimental import pallas as pl
from jax.experimental.pallas import tpu as pltpu
from jax.experimental.pallas import tpu_sc as plsc

N_TOK = 4096
D_IN = 768
D_DICT = 30720
K = 32
CHUNK = 128
N_CHUNKS = D_DICT // CHUNK
BLK_J = 512
L = 16

N_WORKERS = 32
ROWS_PER_W = N_TOK // N_WORKERS
CAND_CAP = D_DICT + 2 * L
SEL_CAP = 256


def _encode_body(nrows, x_ref, we_ref, be_ref, acts_ref, cm_ref):
    xc = x_ref[...]
    pre = jax.lax.dot_general(
        xc, we_ref[...], (((1,), (0,)), ((), ())),
        preferred_element_type=jnp.float32,
    )
    acts = jnp.maximum(pre + be_ref[...], 0.0)
    acts_ref[...] = acts
    cm = jnp.max(acts.reshape(nrows, BLK_J // CHUNK, CHUNK), axis=2)
    cm_ref[...] = cm.reshape(1, nrows, BLK_J // CHUNK)


def _encode(x_cent, W_enc, b_enc, nrows):
    nj = D_DICT // BLK_J
    return pl.pallas_call(
        functools.partial(_encode_body, nrows),
        grid=(nj,),
        in_specs=[
            pl.BlockSpec((nrows, D_IN), lambda j: (0, 0)),
            pl.BlockSpec((D_IN, BLK_J), lambda j: (0, j)),
            pl.BlockSpec((1, BLK_J), lambda j: (0, j)),
        ],
        out_specs=[
            pl.BlockSpec((nrows, BLK_J), lambda j: (0, j)),
            pl.BlockSpec((1, nrows, BLK_J // CHUNK), lambda j: (j, 0, 0)),
        ],
        out_shape=[
            jax.ShapeDtypeStruct((nrows, D_DICT), jnp.float32),
            jax.ShapeDtypeStruct((D_DICT // BLK_J, nrows, BLK_J // CHUNK),
                                 jnp.float32),
        ],
        compiler_params=pltpu.CompilerParams(
            dimension_semantics=("arbitrary",),
        ),
    )(x_cent, W_enc, b_enc)


def _iota16():
    return lax.iota(jnp.int32, L)


def _scal(x):
    return x if x.ndim == 0 else jnp.max(x)


def _extract_i32(vec, lane):
    return jnp.sum(jnp.where(_iota16() == lane, vec, 0))


def _extract_f32(vec, lane):
    return jnp.sum(jnp.where(_iota16() == lane, vec, jnp.float32(0)))


def _sc_body(rows_per_w, acts2d, cm, wdec, out,
             cma_v, cmb_v, cidx_v, cidx2d_v, chunk_v, cval_v, cgi_v,
             selva_v, selvb_v, selia_v, selib_v, wrowsa_v, wrowsb_v,
             outa_v, outb_v, sem, sem2, semc, semo):
    w = lax.axis_index("s") * 2 + lax.axis_index("c")
    lane0 = _iota16() == 0
    r0 = w * rows_per_w
    inf_v = jnp.full((L,), jnp.inf, jnp.float32)
    pltpu.sync_copy(cm.at[r0], cma_v)

    ga = (selva_v, selia_v, wrowsa_v, outa_v)
    gb = (selvb_v, selib_v, wrowsb_v, outb_v)

    def wdec_gather(g):
        selv, seli, wrows, _ = g
        return pltpu.make_async_copy(wdec.at[seli.at[pl.ds(0, K)]], wrows,
                                     sem2)

    def decode_row(g, rt):
        selv, seli, wrows, outb = g
        wdec_gather(g).wait()

        def dec_step(j, acc):
            jg = (j // L) * L
            val = _extract_f32(selv[pl.ds(jg, L)], j - jg)
            vvec = jnp.full((L,), val, jnp.float32)
            return tuple(acc[u] + vvec * wrows[j, pl.ds(u * L, L)]
                         for u in range(D_IN // L))

        acc0 = tuple(jnp.zeros((L,), jnp.float32) for _ in range(D_IN // L))
        acc = lax.fori_loop(0, K, dec_step, acc0)
        for u in range(D_IN // L):
            outb[pl.ds(u * L, L)] = acc[u]
        pltpu.async_copy(outb, out.at[rt], semo)

    def one_row(rl, cm_cur, cm_nxt, g_cur, g_prv):
        r = r0 + rl
        rn = r0 + jnp.minimum(rl + 1, rows_per_w - 1)
        pltpu.async_copy(cm.at[rn], cm_nxt, semc)
        cms = [cm_cur[pl.ds(i * L, L)] for i in range(N_CHUNKS // L)]

        def search_step(i, sbits):
            trial = jnp.bitwise_or(sbits, jnp.left_shift(jnp.int32(1), 30 - i))
            tvec = jnp.full((L,), lax.bitcast_convert_type(trial, jnp.float32),
                            jnp.float32)
            cnt = jnp.zeros((L,), jnp.int32)
            for c in cms:
                cnt = cnt + jnp.where(c >= tvec, 1, 0)
            return jnp.where(jnp.sum(cnt) >= K, trial, sbits)

        sbits = lax.fori_loop(0, 20, search_step, jnp.int32(0))
        svec = jnp.full((L,), lax.bitcast_convert_type(sbits, jnp.float32),
                        jnp.float32)

        base = r * N_CHUNKS
        nsel = jnp.int32(0)
        for i, c in enumerate(cms):
            m = c >= svec
            ids = base + i * L + _iota16()
            plsc.store_compressed(cidx_v.at[pl.ds(nsel, L)], ids, mask=m)
            nsel = nsel + jnp.sum(jnp.where(m, 1, 0))
        for i in range(4):
            cidx_v[pl.ds(nsel + i * L, L)] = base + i * L + _iota16()

        @pl.when(nsel > 64)
        def _():
            for i in range(SEL_CAP // L):
                ids = jnp.minimum(i * L + _iota16(), N_CHUNKS - 1)
                cidx_v[pl.ds(i * L, L)] = base + ids

        nsel2 = jnp.where(nsel > 64, jnp.int32(N_CHUNKS), nsel)
        ngrp = (nsel2 + 63) // 64

        for i in range(SEL_CAP // L):
            cidx2d_v[i // 4, pl.ds((i % 4) * L, L)] = cidx_v[pl.ds(i * L, L)]

        def extract_grp(g, cnt):
            pltpu.async_copy(acts2d.at[cidx2d_v.at[g]], chunk_v, sem).wait()
            kmax = jnp.minimum(nsel2 - g * 64, 64)

            def extract_slot(k, cnt):
                kglob = g * 64 + k
                kg = (kglob // L) * L
                cvec = cidx_v[pl.ds(kg, L)]
                cloc = _extract_i32(cvec, kglob - kg) - base
                gbase = jnp.full((L,), cloc * CHUNK, jnp.int32)
                for u in range(CHUNK // L):
                    v = chunk_v[k, pl.ds(u * L, L)]
                    m = v >= svec
                    gi = gbase + (u * L) + _iota16()
                    plsc.store_compressed(cval_v.at[pl.ds(cnt, L)], v, mask=m)
                    plsc.store_compressed(cgi_v.at[pl.ds(cnt, L)], gi, mask=m)
                    cnt = cnt + jnp.sum(jnp.where(m, 1, 0))
                return cnt

            return lax.fori_loop(0, kmax, extract_slot, cnt)

        cnt = lax.fori_loop(0, ngrp, extract_grp, jnp.int32(0))
        cval_v[pl.ds(cnt, L)] = inf_v
        cval_v[pl.ds(cnt + L, L)] = inf_v
        nv = jnp.maximum((cnt + L - 1) // L, 2)

        def discard_step(_, __):
            def colmin_step(v, acc):
                return jnp.minimum(acc, cval_v[pl.ds(v * L, L)])

            colmin = lax.fori_loop(1, nv, colmin_step, cval_v[pl.ds(0, L)])
            t = jnp.min(colmin)
            tvec = jnp.full((L,), t, jnp.float32)

            def find_cond(st):
                v, found, _ = st
                return jnp.logical_and(v >= 0, found == 0)

            def find_step(st):
                v, found, pos = st
                m = cval_v[pl.ds(v * L, L)] <= tvec
                hit = jnp.sum(jnp.where(m, 1, 0)) > 0
                lastlane = jnp.max(jnp.where(m, _iota16(), -1))
                pos = jnp.where(hit, v * L + lastlane, pos)
                return v - 1, jnp.where(hit, 1, found).astype(jnp.int32), pos

            _, _, pos = lax.while_loop(
                find_cond, find_step, (nv - 1, jnp.int32(0), jnp.int32(0)))
            plsc.store_scatter(cval_v, [jnp.full((L,), pos, jnp.int32)],
                               inf_v, mask=lane0)
            return 0

        lax.fori_loop(0, cnt - K, discard_step, 0)

        selv, seli, _, _ = g_cur

        def collect_step(v, scnt):
            vals = cval_v[pl.ds(v * L, L)]
            m = vals < inf_v
            plsc.store_compressed(selv.at[pl.ds(scnt, L)], vals, mask=m)
            plsc.store_compressed(seli.at[pl.ds(scnt, L)],
                                  cgi_v[pl.ds(v * L, L)], mask=m)
            return scnt + jnp.sum(jnp.where(m, 1, 0))

        lax.fori_loop(0, nv, collect_step, jnp.int32(0))
        wdec_gather(g_cur).start()

        @pl.when(rl >= 3)
        def _():
            pltpu.make_async_copy(g_prv[3], out.at[r], semo).wait()

        @pl.when(rl >= 1)
        def _():
            decode_row(g_prv, r - 1)

        pltpu.make_async_copy(cm.at[rn], cm_nxt, semc).wait()

    def pair_body(rp, _):
        rl = rp * 2
        one_row(rl, cma_v, cmb_v, ga, gb)
        one_row(rl + 1, cmb_v, cma_v, gb, ga)
        return 0

    lax.fori_loop(0, rows_per_w // 2, pair_body, 0)
    pltpu.make_async_copy(gb[3], out.at[r0], semo).wait()
    decode_row(gb, r0 + rows_per_w - 1)
    pltpu.make_async_copy(ga[3], out.at[r0], semo).wait()
    pltpu.make_async_copy(gb[3], out.at[r0], semo).wait()


@functools.partial(jax.jit, static_argnames=("nrows",))
def _sc_select_decode(acts2d, cm, W_dec, nrows=N_TOK):
    kern = pl.kernel(
        functools.partial(_sc_body, nrows // N_WORKERS),
        out_type=jax.ShapeDtypeStruct((nrows, D_IN), jnp.float32),
        mesh=plsc.VectorSubcoreMesh(core_axis_name="c", subcore_axis_name="s"),
        scratch_types=[
            pltpu.VMEM((N_CHUNKS,), jnp.float32),
            pltpu.VMEM((N_CHUNKS,), jnp.float32),
            pltpu.VMEM((SEL_CAP + L,), jnp.int32),
            pltpu.VMEM((4, 64), jnp.int32),
            pltpu.VMEM((64, CHUNK), jnp.float32),
            pltpu.VMEM((CAND_CAP,), jnp.float32),
            pltpu.VMEM((CAND_CAP,), jnp.int32),
            pltpu.VMEM((K + L,), jnp.float32),
            pltpu.VMEM((K + L,), jnp.float32),
            pltpu.VMEM((K + L,), jnp.int32),
            pltpu.VMEM((K + L,), jnp.int32),
            pltpu.VMEM((K, D_IN), jnp.float32),
            pltpu.VMEM((K, D_IN), jnp.float32),
            pltpu.VMEM((D_IN,), jnp.float32),
            pltpu.VMEM((D_IN,), jnp.float32),
            pltpu.SemaphoreType.DMA,
            pltpu.SemaphoreType.DMA,
            pltpu.SemaphoreType.DMA,
            pltpu.SemaphoreType.DMA,
        ],
        compiler_params=pltpu.CompilerParams(needs_layout_passes=False),
    )
    return kern(acts2d, cm, W_dec)


N_PIPE = 4
ROWS_BLK = N_TOK // N_PIPE


def kernel(x, W_enc, W_dec, b_enc, b_dec):
    x_cent = x - b_dec[None, :]
    b_enc2 = b_enc.reshape(1, D_DICT)
    recons = []
    for b in range(N_PIPE):
        xb = lax.dynamic_slice_in_dim(x_cent, b * ROWS_BLK, ROWS_BLK, axis=0)
        acts, cm3 = _encode(xb, W_enc, b_enc2, ROWS_BLK)
        cm = cm3.transpose(1, 0, 2).reshape(ROWS_BLK, N_CHUNKS)
        acts2d = acts.reshape(ROWS_BLK * N_CHUNKS, CHUNK)
        recons.append(_sc_select_decode(acts2d, cm, W_dec, nrows=ROWS_BLK))
    return jnp.concatenate(recons, axis=0) + b_dec[None, :]
